# baseline (speedup 1.0000x reference)
import jax
import jax.numpy as jnp
from jax import lax
from jax.experimental import pallas as pl
from jax.experimental.pallas import tpu as pltpu
from jax.experimental.pallas import tpu_sc as plsc

_B = 32
_N = 8192
_L = 16
_NCH = _N // _L
_RADIX = 256
_HIST = _RADIX * _L


def _tf_pair(k1, k2, x1):
    c = jnp.int32(0x1BD11BDA)
    ks = (k1, k2, k1 ^ k2 ^ c)

    def rotl(v, r):
        return lax.shift_left(v, jnp.int32(r)) | lax.shift_right_logical(
            v, jnp.int32(32 - r))

    x0 = ks[0]
    x1 = x1 + ks[1]
    rots = ((13, 15, 26, 6), (17, 29, 16, 24))
    for i in range(5):
        for r in rots[i % 2]:
            x0 = x0 + x1
            x1 = rotl(x1, r)
            x1 = x1 ^ x0
        x0 = x0 + ks[(i + 1) % 3]
        x1 = x1 + ks[(i + 2) % 3] + jnp.int32(i + 1)
    return x0, x1


def _tf_bits(k1, k2, cnt):
    o0, o1 = _tf_pair(k1, k2, cnt)
    return o0 ^ o1


def _sc_body(pa_hbm, re_hbm, ka, kb, pa, pb, hist):
    wid = lax.axis_index("s") * 2 + lax.axis_index("c")
    lane = lax.iota(jnp.int32, _L)
    lane_nch = lane * _NCH

    zv = jnp.zeros((_L,), jnp.int32)
    b1, b2 = _tf_pair(zv, zv + jnp.int32(42), zv + wid)
    n1, n2 = _tf_pair(b1, b2, zv)
    s1a, s1b = _tf_pair(b1, b2, zv + jnp.int32(1))
    s2a, s2b = _tf_pair(n1, n2, zv + jnp.int32(1))

    def phys(pos):
        return lax.shift_left(pos & jnp.int32(_NCH - 1), jnp.int32(4)) | (
            lax.shift_right_logical(pos, jnp.int32(9)))

    def gen_keys(k1, k2):

        def gen(tt, _):
            for u in range(4):
                t = tt * 4 + u
                ka[pl.ds(t * _L, _L)] = _tf_bits(k1, k2, lane_nch + t)
            return 0

        lax.fori_loop(0, _NCH // 4, gen, 0)

    def radix_pass(src_k, src_p, dst_k, dst_p, shift, first, store_keys,
                   final):
        zeros = jnp.zeros((_L,), jnp.int32)
        ones = jnp.ones((_L,), jnp.int32)
        one = jnp.int32(1)
        zero = jnp.int32(0)
        sh = jnp.int32(shift)
        mask = jnp.int32(0xFF)

        def z(jj, _):
            for u in range(8):
                hist[pl.ds((jj * 8 + u) * _L, _L)] = zeros
            return 0

        lax.fori_loop(0, _HIST // _L // 8, z, 0)

        def cnt(tt, _):
            ks = [src_k[pl.ds((tt * 8 + u) * _L, _L)] for u in range(8)]
            hs = [(lax.shift_right_logical(k, sh) & mask) * _L + lane
                  for k in ks]
            for h in hs:
                plsc.addupdate_scatter(hist, [h], ones)
            return 0

        lax.fori_loop(0, _NCH // 8, cnt, 0)

        def scn(jj, carry):
            vs = [hist[pl.ds((jj * 8 + u) * _L, _L)] for u in range(8)]
            incs = [plsc.cumsum(v) for v in vs]
            for u in range(8):
                hist[pl.ds((jj * 8 + u) * _L, _L)] = incs[u] - vs[u] + carry
                carry = carry + incs[u][15]
            return carry

        lax.fori_loop(0, _HIST // _L // 8, scn, jnp.int32(0))

        U = 4

        def prm(tt, _):
            ks, ps, hs = [], [], []
            for u in range(U):
                t = tt * U + u
                ks.append(src_k[pl.ds(t * _L, _L)])
                ps.append((lane_nch + t) if first
                          else src_p[pl.ds(t * _L, _L)])
                hs.append((lax.shift_right_logical(ks[u], sh) & mask) * _L
                          + lane)
            bases = [plsc.load_gather(hist, [h]) for h in hs]
            for h in hs:
                plsc.addupdate_scatter(hist, [h], ones)
            poss = []
            for u in range(U):
                pos = bases[u]
                for up in range(u):
                    pos = pos + jnp.where(hs[up] == hs[u], one, zero)
                poss.append(pos)
            for u in range(U):
                wpos = poss[u] if final else phys(poss[u])
                if store_keys:
                    plsc.store_scatter(dst_k, [wpos], ks[u])
                plsc.store_scatter(dst_p, [wpos], ps[u])
            return 0

        lax.fori_loop(0, _NCH // U, prm, 0)

    with jax.named_scope("gen1"):
        gen_keys(s1a, s1b)
    with jax.named_scope("sort1"):
        radix_pass(ka, None, kb, pb, 0, True, True, False)
        radix_pass(kb, pb, ka, pa, 8, False, True, False)
        radix_pass(ka, pa, kb, pb, 16, False, True, False)
        radix_pass(kb, pb, ka, pa, 24, False, False, False)
    with jax.named_scope("gen2"):
        gen_keys(s2a, s2b)
    with jax.named_scope("sort2"):
        radix_pass(ka, pa, kb, pb, 0, False, True, False)
        radix_pass(kb, pb, ka, pa, 8, False, True, False)
        radix_pass(ka, pa, kb, pb, 16, False, True, False)
        radix_pass(kb, pb, ka, pa, 24, False, False, True)

    def inv(tt, _):
        vs = [pa[pl.ds((tt * 8 + u) * _L, _L)] for u in range(8)]
        for u in range(8):
            plsc.store_scatter(kb, [vs[u]], lane + (tt * 8 + u) * _L)
        return 0

    lax.fori_loop(0, _NCH // 8, inv, 0)
    pltpu.sync_copy(pa, pa_hbm.at[wid])
    pltpu.sync_copy(kb, re_hbm.at[wid])


def _make_kernel(interpret=False):
    mesh = plsc.VectorSubcoreMesh(core_axis_name="c", subcore_axis_name="s",
                                  num_cores=2, num_subcores=16)
    return pl.kernel(
        _sc_body,
        out_type=(jax.ShapeDtypeStruct((_B, _N), jnp.int32),
                  jax.ShapeDtypeStruct((_B, _N), jnp.int32)),
        mesh=mesh,
        compiler_params=pltpu.CompilerParams(needs_layout_passes=False),
        scratch_types=[
            pltpu.VMEM((_N,), jnp.int32),
            pltpu.VMEM((_N,), jnp.int32),
            pltpu.VMEM((_N,), jnp.int32),
            pltpu.VMEM((_N,), jnp.int32),
            pltpu.VMEM((_HIST,), jnp.int32),
        ],
        interpret=interpret,
    )


def kernel(z):
    del z
    idx_pa, idx_re = _make_kernel()()
    return idx_pa, idx_re


# speedup vs baseline: 1.0532x; 1.0532x over previous
import jax
import jax.numpy as jnp
from jax import lax
from jax.experimental import pallas as pl
from jax.experimental.pallas import tpu as pltpu
from jax.experimental.pallas import tpu_sc as plsc

_B = 32
_N = 8192
_L = 16
_NCH = _N // _L
_RADIX = 256
_HIST = _RADIX * _L


def _tf_pair(k1, k2, x1):
    c = jnp.int32(0x1BD11BDA)
    ks = (k1, k2, k1 ^ k2 ^ c)

    def rotl(v, r):
        return lax.shift_left(v, jnp.int32(r)) | lax.shift_right_logical(
            v, jnp.int32(32 - r))

    x0 = ks[0]
    x1 = x1 + ks[1]
    rots = ((13, 15, 26, 6), (17, 29, 16, 24))
    for i in range(5):
        for r in rots[i % 2]:
            x0 = x0 + x1
            x1 = rotl(x1, r)
            x1 = x1 ^ x0
        x0 = x0 + ks[(i + 1) % 3]
        x1 = x1 + ks[(i + 2) % 3] + jnp.int32(i + 1)
    return x0, x1


def _tf_bits(k1, k2, cnt):
    o0, o1 = _tf_pair(k1, k2, cnt)
    return o0 ^ o1


def _sc_body(pa_hbm, re_hbm, ka, kb, pa, pb, hist):
    wid = lax.axis_index("s") * 2 + lax.axis_index("c")
    lane = lax.iota(jnp.int32, _L)
    lane_nch = lane * _NCH

    zv = jnp.zeros((_L,), jnp.int32)
    b1, b2 = _tf_pair(zv, zv + jnp.int32(42), zv + wid)
    n1, n2 = _tf_pair(b1, b2, zv)
    s1a, s1b = _tf_pair(b1, b2, zv + jnp.int32(1))
    s2a, s2b = _tf_pair(n1, n2, zv + jnp.int32(1))

    def phys(pos):
        return lax.shift_left(pos & jnp.int32(_NCH - 1), jnp.int32(4)) | (
            lax.shift_right_logical(pos, jnp.int32(9)))

    def gen_keys(k1, k2):

        def gen(tt, _):
            for u in range(4):
                t = tt * 4 + u
                ka[pl.ds(t * _L, _L)] = _tf_bits(k1, k2, lane_nch + t)
            return 0

        lax.fori_loop(0, _NCH // 4, gen, 0)

    def radix_pass(src_k, src_p, dst_k, dst_p, shift, first, store_keys,
                   final):
        zeros = jnp.zeros((_L,), jnp.int32)
        ones = jnp.ones((_L,), jnp.int32)
        one = jnp.int32(1)
        zero = jnp.int32(0)
        sh = jnp.int32(shift)
        mask = jnp.int32(0xFF)

        def z(jj, _):
            for u in range(8):
                hist[pl.ds((jj * 8 + u) * _L, _L)] = zeros
            return 0

        lax.fori_loop(0, _HIST // _L // 8, z, 0)

        def cload(tt):
            ks = [src_k[pl.ds((tt * 8 + u) * _L, _L)] for u in range(8)]
            return tuple((lax.shift_right_logical(k, sh) & mask) * _L + lane
                         for k in ks)

        def cnt(tt, carry):
            nxt = cload(tt + 1)
            for h in carry:
                plsc.addupdate_scatter(hist, [h], ones)
            return nxt

        tail = lax.fori_loop(0, _NCH // 8 - 1, cnt, cload(0))
        for h in tail:
            plsc.addupdate_scatter(hist, [h], ones)

        def scn(jj, carry):
            vs = [hist[pl.ds((jj * 8 + u) * _L, _L)] for u in range(8)]
            incs = [plsc.cumsum(v) for v in vs]
            for u in range(8):
                hist[pl.ds((jj * 8 + u) * _L, _L)] = incs[u] - vs[u] + carry
                carry = carry + incs[u][15]
            return carry

        lax.fori_loop(0, _HIST // _L // 8, scn, jnp.int32(0))

        U = 4

        def pload(tt):
            ks, ps, hs = [], [], []
            for u in range(U):
                t = tt * U + u
                ks.append(src_k[pl.ds(t * _L, _L)])
                ps.append((lane_nch + t) if first
                          else src_p[pl.ds(t * _L, _L)])
                hs.append((lax.shift_right_logical(ks[u], sh) & mask) * _L
                          + lane)
            return tuple(ks) + tuple(ps) + tuple(hs)

        def pscatter(vals):
            ks, ps, hs = vals[:U], vals[U:2 * U], vals[2 * U:]
            bases = [plsc.load_gather(hist, [h]) for h in hs]
            for h in hs:
                plsc.addupdate_scatter(hist, [h], ones)
            poss = []
            for u in range(U):
                pos = bases[u]
                for up in range(u):
                    pos = pos + jnp.where(hs[up] == hs[u], one, zero)
                poss.append(pos)
            for u in range(U):
                wpos = poss[u] if final else phys(poss[u])
                if store_keys:
                    plsc.store_scatter(dst_k, [wpos], ks[u])
                plsc.store_scatter(dst_p, [wpos], ps[u])

        def prm(tt, carry):
            nxt = pload(tt + 1)
            pscatter(carry)
            return nxt

        tail = lax.fori_loop(0, _NCH // U - 1, prm, pload(0))
        pscatter(tail)

    with jax.named_scope("gen1"):
        gen_keys(s1a, s1b)
    with jax.named_scope("sort1"):
        radix_pass(ka, None, kb, pb, 0, True, True, False)
        radix_pass(kb, pb, ka, pa, 8, False, True, False)
        radix_pass(ka, pa, kb, pb, 16, False, True, False)
        radix_pass(kb, pb, ka, pa, 24, False, False, False)
    with jax.named_scope("gen2"):
        gen_keys(s2a, s2b)
    with jax.named_scope("sort2"):
        radix_pass(ka, pa, kb, pb, 0, False, True, False)
        radix_pass(kb, pb, ka, pa, 8, False, True, False)
        radix_pass(ka, pa, kb, pb, 16, False, True, False)
        radix_pass(kb, pb, ka, pa, 24, False, False, True)

    def inv(tt, _):
        vs = [pa[pl.ds((tt * 8 + u) * _L, _L)] for u in range(8)]
        for u in range(8):
            plsc.store_scatter(kb, [vs[u]], lane + (tt * 8 + u) * _L)
        return 0

    lax.fori_loop(0, _NCH // 8, inv, 0)
    pltpu.sync_copy(pa, pa_hbm.at[wid])
    pltpu.sync_copy(kb, re_hbm.at[wid])


def _make_kernel(interpret=False):
    mesh = plsc.VectorSubcoreMesh(core_axis_name="c", subcore_axis_name="s",
                                  num_cores=2, num_subcores=16)
    return pl.kernel(
        _sc_body,
        out_type=(jax.ShapeDtypeStruct((_B, _N), jnp.int32),
                  jax.ShapeDtypeStruct((_B, _N), jnp.int32)),
        mesh=mesh,
        compiler_params=pltpu.CompilerParams(needs_layout_passes=False),
        scratch_types=[
            pltpu.VMEM((_N,), jnp.int32),
            pltpu.VMEM((_N,), jnp.int32),
            pltpu.VMEM((_N,), jnp.int32),
            pltpu.VMEM((_N,), jnp.int32),
            pltpu.VMEM((_HIST,), jnp.int32),
        ],
        interpret=interpret,
    )


def kernel(z):
    del z
    idx_pa, idx_re = _make_kernel()()
    return idx_pa, idx_re
